# baseline (speedup 1.0000x reference)
import jax
import jax.numpy as jnp
from jax.experimental import pallas as pl
from jax.experimental.pallas import tpu as pltpu

_NUM_CLASSES = 80
_NA = _NUM_CLASSES + 5
_A = 3
_H = 76
_W = 76
_XY_SCALE = 1.05
_XY_OFF = 0.5 * (_XY_SCALE - 1.0)
_ANCHOR_HALF = [(10.0 / 608.0 * 0.5, 13.0 / 608.0 * 0.5),
                (16.0 / 608.0 * 0.5, 30.0 / 608.0 * 0.5),
                (33.0 / 608.0 * 0.5, 23.0 / 608.0 * 0.5)]


def _sigmoid(v):
    return 1.0 / (1.0 + jnp.exp(-v))


def _compute(in_slot, out_slot, scr):
    gx = jax.lax.broadcasted_iota(jnp.int32, (1, _H, _W), 2).astype(jnp.float32)
    gy = jax.lax.broadcasted_iota(jnp.int32, (1, _H, _W), 1).astype(jnp.float32)
    inv_g = 1.0 / float(_W)

    for a in range(_A):
        v = in_slot[a * _NA:(a + 1) * _NA]
        cx = (_sigmoid(v[0:1]) * _XY_SCALE - _XY_OFF + gx) * inv_g
        cy = (_sigmoid(v[1:2]) * _XY_SCALE - _XY_OFF + gy) * inv_g
        hw = jnp.exp(v[2:3]) * _ANCHOR_HALF[a][0]
        hh = jnp.exp(v[3:4]) * _ANCHOR_HALF[a][1]
        rest = _sigmoid(v[4:_NA])
        pre = jnp.concatenate(
            [cx - hw, cy - hh, cx + hw, cy + hh, rest], axis=0)
        scr[a] = pre.reshape(_NA * _H, _W)
        for h in range(_H):
            t = jnp.transpose(scr[a, h::_H, :])
            base = h * (_W * _A) + a
            out_slot[base:base + _W * _A:_A, :] = t


def _decode_kernel(x_hbm, o_hbm, in_bufs, out_bufs, scr, in_sems, out_sems):
    b = pl.program_id(0)
    nb = pl.num_programs(0)
    cur = jax.lax.rem(b, 2)

    @pl.when(b == 0)
    def _():
        pltpu.make_async_copy(x_hbm.at[0], in_bufs.at[0], in_sems.at[0]).start()
        pltpu.make_async_copy(x_hbm.at[1], in_bufs.at[1], in_sems.at[1]).start()

    pltpu.make_async_copy(x_hbm.at[b], in_bufs.at[cur], in_sems.at[cur]).wait()

    @pl.when(b >= 2)
    def _():
        pltpu.make_async_copy(
            out_bufs.at[cur], o_hbm.at[b], out_sems.at[cur]).wait()

    _compute(in_bufs.at[cur], out_bufs.at[cur], scr)

    pltpu.make_async_copy(out_bufs.at[cur], o_hbm.at[b], out_sems.at[cur]).start()

    @pl.when(b < nb - 2)
    def _():
        pltpu.make_async_copy(
            x_hbm.at[b + 2], in_bufs.at[cur], in_sems.at[cur]).start()

    @pl.when(b == nb - 1)
    def _():
        pltpu.make_async_copy(
            out_bufs.at[1 - cur], o_hbm.at[nb - 2], out_sems.at[1 - cur]).wait()
        pltpu.make_async_copy(
            out_bufs.at[cur], o_hbm.at[nb - 1], out_sems.at[cur]).wait()


def kernel(x):
    B = x.shape[0]
    out = pl.pallas_call(
        _decode_kernel,
        grid=(B,),
        in_specs=[pl.BlockSpec(memory_space=pl.ANY)],
        out_specs=pl.BlockSpec(memory_space=pl.ANY),
        out_shape=jax.ShapeDtypeStruct((B, _H * _W * _A, _NA), jnp.float32),
        scratch_shapes=[
            pltpu.VMEM((2, _A * _NA, _H, _W), jnp.float32),
            pltpu.VMEM((2, _H * _W * _A, _NA), jnp.float32),
            pltpu.VMEM((_A, _NA * _H, _W), jnp.float32),
            pltpu.SemaphoreType.DMA((2,)),
            pltpu.SemaphoreType.DMA((2,)),
        ],
        compiler_params=pltpu.CompilerParams(
            dimension_semantics=("arbitrary",),
            vmem_limit_bytes=60 * 1024 * 1024),
    )(x)
    return out
